# baseline (speedup 1.0000x reference)
import functools

import jax
import jax.numpy as jnp
from jax import lax
from jax.experimental import pallas as pl
from jax.experimental.pallas import tpu as pltpu
from jax.experimental.pallas import tpu_sc as plsc

T = 4096
D = 2048
E = 16
TOPK = 8
F = 1408
BLK = 256
NSLOT = T * TOPK + E * BLK
NB = NSLOT // BLK
NBP = 256
NW = 32
TPW = T // NW
PPW = TPW * TOPK
SPW = NSLOT // NW


def _router_body(x_ref, w_ref, w16_ref, prefix_ref, counts_ref, cnt_ref):
    i = pl.program_id(0)
    x = x_ref[...]
    logits = lax.dot_general(x, w_ref[...], (((1,), (1,)), ((), ())),
                             preferred_element_type=jnp.float32)
    m = jnp.max(logits, axis=1, keepdims=True)
    ex = jnp.exp(logits - m)
    probs = ex / jnp.sum(ex, axis=1, keepdims=True)

    lane = lax.broadcasted_iota(jnp.int32, probs.shape, 1)
    sel = jnp.zeros(probs.shape, dtype=jnp.bool_)
    for _ in range(TOPK):
        cur = jnp.where(sel, -jnp.inf, probs)
        mx = jnp.max(cur, axis=1, keepdims=True)
        ismax = jnp.logical_and(cur == mx, jnp.logical_not(sel))
        first = jnp.min(jnp.where(ismax, lane, E), axis=1, keepdims=True)
        sel = jnp.logical_or(sel, lane == first)
    kept = jnp.where(sel, probs, 0.0)
    w16_ref[...] = kept / jnp.sum(kept, axis=1, keepdims=True)

    @pl.when(i == 0)
    def _():
        cnt_ref[...] = jnp.zeros_like(cnt_ref)

    self = sel.astype(jnp.float32)
    bt = self.shape[0]
    r = lax.broadcasted_iota(jnp.int32, (bt, bt), 0)
    c = lax.broadcasted_iota(jnp.int32, (bt, bt), 1)
    lt = (r > c).astype(jnp.float32)
    within = lax.dot_general(lt, self, (((1,), (0,)), ((), ())),
                             preferred_element_type=jnp.float32)
    prefix_ref[...] = (within + cnt_ref[...]).astype(jnp.int32)
    cnt_ref[...] += jnp.sum(self, axis=0, keepdims=True)
    counts_ref[...] = cnt_ref[...].astype(jnp.int32)


def _router_dispatch(x, router_w):
    bt = 512
    return pl.pallas_call(
        _router_body,
        grid=(T // bt,),
        in_specs=[
            pl.BlockSpec((bt, D), lambda i: (i, 0)),
            pl.BlockSpec((E, D), lambda i: (0, 0)),
        ],
        out_specs=[
            pl.BlockSpec((bt, E), lambda i: (i, 0)),
            pl.BlockSpec((bt, E), lambda i: (i, 0)),
            pl.BlockSpec((1, E), lambda i: (0, 0)),
        ],
        out_shape=[
            jax.ShapeDtypeStruct((T, E), jnp.float32),
            jax.ShapeDtypeStruct((T, E), jnp.int32),
            jax.ShapeDtypeStruct((1, E), jnp.int32),
        ],
        scratch_shapes=[pltpu.VMEM((1, E), jnp.float32)],
    )(x, router_w)


def _a2_body(prefix_ref, w16_ref, counts_ref, pos8_ref, w8_ref, be_ref):
    eye = (lax.broadcasted_iota(jnp.int32, (E, E), 0)
           == lax.broadcasted_iota(jnp.int32, (E, E), 1)).astype(jnp.float32)
    lt_incl = (lax.broadcasted_iota(jnp.int32, (E, E), 0)
               >= lax.broadcasted_iota(jnp.int32, (E, E), 1)).astype(
                   jnp.float32)
    lt_strict = (lax.broadcasted_iota(jnp.int32, (E, E), 0)
                 < lax.broadcasted_iota(jnp.int32, (E, E), 1)).astype(
                     jnp.float32)

    counts_row = counts_ref[...].astype(jnp.float32)
    counts_col = lax.dot_general(eye, counts_row,
                                 (((1,), (1,)), ((), ())),
                                 preferred_element_type=jnp.float32)
    aligned_col = jnp.ceil(counts_col / BLK) * BLK
    incl_col = lax.dot_general(lt_incl, aligned_col,
                               (((1,), (0,)), ((), ())),
                               preferred_element_type=jnp.float32)
    offs_col = incl_col - aligned_col
    offs_row = lax.dot_general(offs_col, eye, (((0,), (0,)), ((), ())),
                               preferred_element_type=jnp.float32)

    w16 = w16_ref[...]
    sel = w16 > 0.0
    sel_f = sel.astype(jnp.float32)
    pos16 = prefix_ref[...].astype(jnp.float32) + offs_row
    rk = lax.dot_general(sel_f, lt_strict, (((1,), (0,)), ((), ())),
                         preferred_element_type=jnp.float32)

    bt = w16.shape[0]
    kio = lax.broadcasted_iota(jnp.int32, (bt, TOPK), 1).astype(jnp.float32)
    pos8 = jnp.zeros((bt, TOPK), jnp.float32)
    w8 = jnp.zeros((bt, TOPK), jnp.float32)
    for e in range(E):
        cond = jnp.logical_and(rk[:, e:e + 1] == kio, sel[:, e:e + 1])
        pos8 = jnp.where(cond, pos16[:, e:e + 1], pos8)
        w8 = jnp.where(cond, w16[:, e:e + 1], w8)
    pos8_ref[...] = pos8.astype(jnp.int32)
    w8_ref[...] = w8

    blk_mat = lax.broadcasted_iota(jnp.int32, (E, NBP), 1).astype(
        jnp.float32) * BLK
    ge = (blk_mat >= incl_col).astype(jnp.float32)
    ones_row = jnp.ones((1, E), jnp.float32)
    be = lax.dot_general(ones_row, ge, (((1,), (0,)), ((), ())),
                         preferred_element_type=jnp.float32)
    be_ref[...] = jnp.minimum(be, float(E - 1)).astype(jnp.int32)


def _a2(prefix, w16, counts):
    bt = 512
    return pl.pallas_call(
        _a2_body,
        grid=(T // bt,),
        in_specs=[
            pl.BlockSpec((bt, E), lambda i: (i, 0)),
            pl.BlockSpec((bt, E), lambda i: (i, 0)),
            pl.BlockSpec((1, E), lambda i: (0, 0)),
        ],
        out_specs=[
            pl.BlockSpec((bt, TOPK), lambda i: (i, 0)),
            pl.BlockSpec((bt, TOPK), lambda i: (i, 0)),
            pl.BlockSpec((1, NBP), lambda i: (0, 0)),
        ],
        out_shape=[
            jax.ShapeDtypeStruct((T, TOPK), jnp.int32),
            jax.ShapeDtypeStruct((T, TOPK), jnp.float32),
            jax.ShapeDtypeStruct((1, NBP), jnp.int32),
        ],
    )(prefix, w16, counts)


def _scatter_tok_body(pos8_hbm, slot_tok_hbm, idx_v, tok_v, sem):
    wid = lax.axis_index("s") * 2 + lax.axis_index("c")
    base_p = wid * PPW
    pltpu.sync_copy(pos8_hbm.at[pl.ds(base_p, PPW)], idx_v)

    def fill(j, _):
        pair = lax.iota(jnp.int32, 16) + (base_p + j * 16)
        tok_v[pl.ds(j * 16, 16)] = lax.shift_right_logical(pair, 3)
        return 0

    lax.fori_loop(0, PPW // 16, fill, 0)
    pltpu.async_copy(tok_v, slot_tok_hbm.at[idx_v], sem).wait()


def _sc_scatter_tok(pos8_flat):
    mesh = plsc.VectorSubcoreMesh(core_axis_name="c", subcore_axis_name="s")
    kfn = pl.kernel(
        _scatter_tok_body,
        out_type=[jax.ShapeDtypeStruct((NSLOT,), jnp.int32)],
        mesh=mesh,
        scratch_types=[
            pltpu.VMEM((PPW,), jnp.int32),
            pltpu.VMEM((PPW,), jnp.int32),
            pltpu.SemaphoreType.DMA,
        ],
    )
    return kfn(pos8_flat)[0]


def _gather_x_body(x_hbm, slot_tok_hbm, xs_hbm, idxv, buf, sem):
    wid = lax.axis_index("s") * 2 + lax.axis_index("c")
    base = wid * SPW
    pltpu.sync_copy(slot_tok_hbm.at[pl.ds(base, SPW)], idxv)

    def clamp(j, _):
        v = idxv[pl.ds(j * 16, 16)]
        idxv[pl.ds(j * 16, 16)] = jnp.minimum(
            jnp.maximum(v, jnp.int32(0)), jnp.int32(T - 1))
        return 0

    lax.fori_loop(0, SPW // 16, clamp, 0)

    G = 48

    def chunk(c, _):
        pltpu.async_copy(x_hbm.at[idxv.at[pl.ds(c * G, G)]], buf, sem).wait()
        pltpu.sync_copy(buf, xs_hbm.at[pl.ds(base + c * G, G)])
        return 0

    lax.fori_loop(0, SPW // G, chunk, 0)


def _sc_gather_x(x32, slot_tok):
    mesh = plsc.VectorSubcoreMesh(core_axis_name="c", subcore_axis_name="s")
    kfn = pl.kernel(
        _gather_x_body,
        out_type=[jax.ShapeDtypeStruct((NSLOT, D), jnp.float32)],
        mesh=mesh,
        scratch_types=[
            pltpu.VMEM((SPW,), jnp.int32),
            pltpu.VMEM((48, D), jnp.float32),
            pltpu.SemaphoreType.DMA,
        ],
    )
    return kfn(x32, slot_tok)[0]


def _gmm_body(be_ref, xs_ref, wg_ref, wu_ref, wd_ref, y_ref):
    x = xs_ref[...].astype(jnp.bfloat16)
    g = lax.dot_general(x, wg_ref[0], (((1,), (1,)), ((), ())),
                        preferred_element_type=jnp.float32)
    u = lax.dot_general(x, wu_ref[0], (((1,), (1,)), ((), ())),
                        preferred_element_type=jnp.float32)
    h = (g * jax.nn.sigmoid(g) * u).astype(jnp.bfloat16)
    y_ref[...] = lax.dot_general(h, wd_ref[0], (((1,), (1,)), ((), ())),
                                 preferred_element_type=jnp.float32)


def _gmm(block_expert, xs, wg16, wu16, wd16):
    grid_spec = pltpu.PrefetchScalarGridSpec(
        num_scalar_prefetch=1,
        grid=(NB,),
        in_specs=[
            pl.BlockSpec((BLK, D), lambda i, be: (i, 0)),
            pl.BlockSpec((1, F, D), lambda i, be: (be[i], 0, 0)),
            pl.BlockSpec((1, F, D), lambda i, be: (be[i], 0, 0)),
            pl.BlockSpec((1, D, F), lambda i, be: (be[i], 0, 0)),
        ],
        out_specs=pl.BlockSpec((BLK, D), lambda i, be: (i, 0)),
    )
    return pl.pallas_call(
        _gmm_body,
        grid_spec=grid_spec,
        out_shape=jax.ShapeDtypeStruct((NSLOT, D), jnp.float32),
    )(block_expert, xs, wg16, wu16, wd16)


def _gather_y_body(y_hbm, pos8_hbm, yg_hbm, idxv, buf, sem):
    wid = lax.axis_index("s") * 2 + lax.axis_index("c")
    base_p = wid * PPW
    pltpu.sync_copy(pos8_hbm.at[pl.ds(base_p, PPW)], idxv)

    G = 32

    def chunk(c, _):
        pltpu.async_copy(y_hbm.at[idxv.at[pl.ds(c * G, G)]], buf, sem).wait()
        pltpu.sync_copy(buf, yg_hbm.at[pl.ds(base_p + c * G, G)])
        return 0

    lax.fori_loop(0, PPW // G, chunk, 0)


def _sc_gather_y(y, pos8_flat):
    mesh = plsc.VectorSubcoreMesh(core_axis_name="c", subcore_axis_name="s")
    kfn = pl.kernel(
        _gather_y_body,
        out_type=[jax.ShapeDtypeStruct((T * TOPK, D), jnp.float32)],
        mesh=mesh,
        scratch_types=[
            pltpu.VMEM((PPW,), jnp.int32),
            pltpu.VMEM((32, D), jnp.float32),
            pltpu.SemaphoreType.DMA,
        ],
    )
    return kfn(y, pos8_flat)[0]


def _reduce_body(yg_ref, w8_ref, eo_ref):
    w8 = w8_ref[...]
    acc = w8[:, 0:1] * yg_ref[:, 0, :]
    for k in range(1, TOPK):
        acc += w8[:, k:k + 1] * yg_ref[:, k, :]
    eo_ref[...] = acc


def _reduce(yg, w8):
    bt = 256
    return pl.pallas_call(
        _reduce_body,
        grid=(T // bt,),
        in_specs=[
            pl.BlockSpec((bt, TOPK, D), lambda i: (i, 0, 0)),
            pl.BlockSpec((bt, TOPK), lambda i: (i, 0)),
        ],
        out_specs=pl.BlockSpec((bt, D), lambda i: (i, 0)),
        out_shape=jax.ShapeDtypeStruct((T, D), jnp.float32),
    )(yg, w8)


def _shared_body(nf, x_ref, wsg_ref, wsu_ref, wsd_ref, wshg_ref, eo_ref,
                 out_ref, acc_ref):
    j = pl.program_id(1)
    x = x_ref[...]
    g = lax.dot_general(x, wsg_ref[...], (((1,), (1,)), ((), ())),
                        preferred_element_type=jnp.float32)
    u = lax.dot_general(x, wsu_ref[...], (((1,), (1,)), ((), ())),
                        preferred_element_type=jnp.float32)
    h = (g * jax.nn.sigmoid(g) * u).astype(jnp.bfloat16)
    part = lax.dot_general(h, wsd_ref[...], (((1,), (1,)), ((), ())),
                           preferred_element_type=jnp.float32)

    @pl.when(j == 0)
    def _():
        acc_ref[...] = part

    @pl.when(j != 0)
    def _():
        acc_ref[...] += part

    @pl.when(j == nf - 1)
    def _():
        gl = jnp.sum(x.astype(jnp.float32) * wshg_ref[...].astype(jnp.float32),
                     axis=1, keepdims=True)
        out_ref[...] = eo_ref[...] + jax.nn.sigmoid(gl) * acc_ref[...]


def _shared_combine(x16, ws_gate, ws_up, ws_down, w_shared_gate, eo):
    f_sh = ws_gate.shape[0]
    bt = 256
    bf = 1408
    nf = f_sh // bf
    return pl.pallas_call(
        functools.partial(_shared_body, nf),
        grid=(T // bt, nf),
        in_specs=[
            pl.BlockSpec((bt, D), lambda i, j: (i, 0)),
            pl.BlockSpec((bf, D), lambda i, j: (j, 0)),
            pl.BlockSpec((bf, D), lambda i, j: (j, 0)),
            pl.BlockSpec((D, bf), lambda i, j: (0, j)),
            pl.BlockSpec((1, D), lambda i, j: (0, 0)),
            pl.BlockSpec((bt, D), lambda i, j: (i, 0)),
        ],
        out_specs=pl.BlockSpec((bt, D), lambda i, j: (i, 0)),
        out_shape=jax.ShapeDtypeStruct((T, D), jnp.float32),
        scratch_shapes=[pltpu.VMEM((bt, D), jnp.float32)],
    )(x16, ws_gate, ws_up, ws_down, w_shared_gate, eo)


def kernel(hidden_states, router_w, w_gate, w_up, w_down, ws_gate, ws_up,
           ws_down, w_shared_gate):
    b, s, d = hidden_states.shape
    x = hidden_states.reshape(-1, d)

    w16, prefix, counts = _router_dispatch(x, router_w)
    pos8, w8, be = _a2(prefix, w16, counts)
    pos8_flat = pos8.reshape(-1)

    x16 = x.astype(jnp.bfloat16)
    slot_tok = _sc_scatter_tok(pos8_flat)
    xs = _sc_gather_x(x, slot_tok)
    y = _gmm(be.reshape(-1)[:NB], xs, w_gate.astype(jnp.bfloat16),
             w_up.astype(jnp.bfloat16), w_down.astype(jnp.bfloat16))
    yg = _sc_gather_y(y, pos8_flat)
    eo = _reduce(yg.reshape(T, TOPK, D), w8)

    out = _shared_combine(x16, ws_gate.astype(jnp.bfloat16),
                          ws_up.astype(jnp.bfloat16),
                          ws_down.astype(jnp.bfloat16),
                          w_shared_gate.astype(jnp.bfloat16), eo)
    return out.reshape(b, s, d)


# speedup vs baseline: 1.0093x; 1.0093x over previous
import functools

import jax
import jax.numpy as jnp
from jax import lax
from jax.experimental import pallas as pl
from jax.experimental.pallas import tpu as pltpu
from jax.experimental.pallas import tpu_sc as plsc

T = 4096
D = 2048
E = 16
TOPK = 8
F = 1408
BLK = 256
NSLOT = T * TOPK + E * BLK
NB = NSLOT // BLK
NBP = 256
NW = 32
TPW = T // NW
PPW = TPW * TOPK
SPW = NSLOT // NW


def _router_body(x_ref, w_ref, w16_ref, prefix_ref, counts_ref, cnt_ref):
    i = pl.program_id(0)
    x = x_ref[...]
    logits = lax.dot_general(x, w_ref[...], (((1,), (1,)), ((), ())),
                             preferred_element_type=jnp.float32)
    m = jnp.max(logits, axis=1, keepdims=True)
    ex = jnp.exp(logits - m)
    probs = ex / jnp.sum(ex, axis=1, keepdims=True)

    lane = lax.broadcasted_iota(jnp.int32, probs.shape, 1)
    sel = jnp.zeros(probs.shape, dtype=jnp.bool_)
    for _ in range(TOPK):
        cur = jnp.where(sel, -jnp.inf, probs)
        mx = jnp.max(cur, axis=1, keepdims=True)
        ismax = jnp.logical_and(cur == mx, jnp.logical_not(sel))
        first = jnp.min(jnp.where(ismax, lane, E), axis=1, keepdims=True)
        sel = jnp.logical_or(sel, lane == first)
    kept = jnp.where(sel, probs, 0.0)
    w16_ref[...] = kept / jnp.sum(kept, axis=1, keepdims=True)

    @pl.when(i == 0)
    def _():
        cnt_ref[...] = jnp.zeros_like(cnt_ref)

    self = sel.astype(jnp.float32)
    bt = self.shape[0]
    r = lax.broadcasted_iota(jnp.int32, (bt, bt), 0)
    c = lax.broadcasted_iota(jnp.int32, (bt, bt), 1)
    lt = (r > c).astype(jnp.float32)
    within = lax.dot_general(lt, self, (((1,), (0,)), ((), ())),
                             preferred_element_type=jnp.float32)
    prefix_ref[...] = (within + cnt_ref[...]).astype(jnp.int32)
    cnt_ref[...] += jnp.sum(self, axis=0, keepdims=True)
    counts_ref[...] = cnt_ref[...].astype(jnp.int32)


def _router_dispatch(x, router_w):
    bt = 512
    return pl.pallas_call(
        _router_body,
        grid=(T // bt,),
        in_specs=[
            pl.BlockSpec((bt, D), lambda i: (i, 0)),
            pl.BlockSpec((E, D), lambda i: (0, 0)),
        ],
        out_specs=[
            pl.BlockSpec((bt, E), lambda i: (i, 0)),
            pl.BlockSpec((bt, E), lambda i: (i, 0)),
            pl.BlockSpec((1, E), lambda i: (0, 0)),
        ],
        out_shape=[
            jax.ShapeDtypeStruct((T, E), jnp.float32),
            jax.ShapeDtypeStruct((T, E), jnp.int32),
            jax.ShapeDtypeStruct((1, E), jnp.int32),
        ],
        scratch_shapes=[pltpu.VMEM((1, E), jnp.float32)],
    )(x, router_w)


def _a2_body(prefix_ref, w16_ref, counts_ref, pos8_ref, w8_ref, be_ref):
    eye = (lax.broadcasted_iota(jnp.int32, (E, E), 0)
           == lax.broadcasted_iota(jnp.int32, (E, E), 1)).astype(jnp.float32)
    lt_incl = (lax.broadcasted_iota(jnp.int32, (E, E), 0)
               >= lax.broadcasted_iota(jnp.int32, (E, E), 1)).astype(
                   jnp.float32)
    lt_strict = (lax.broadcasted_iota(jnp.int32, (E, E), 0)
                 < lax.broadcasted_iota(jnp.int32, (E, E), 1)).astype(
                     jnp.float32)

    counts_row = counts_ref[...].astype(jnp.float32)
    counts_col = lax.dot_general(eye, counts_row,
                                 (((1,), (1,)), ((), ())),
                                 preferred_element_type=jnp.float32)
    aligned_col = jnp.ceil(counts_col / BLK) * BLK
    incl_col = lax.dot_general(lt_incl, aligned_col,
                               (((1,), (0,)), ((), ())),
                               preferred_element_type=jnp.float32)
    offs_col = incl_col - aligned_col
    offs_row = lax.dot_general(offs_col, eye, (((0,), (0,)), ((), ())),
                               preferred_element_type=jnp.float32)

    w16 = w16_ref[...]
    sel = w16 > 0.0
    sel_f = sel.astype(jnp.float32)
    pos16 = prefix_ref[...].astype(jnp.float32) + offs_row
    rk = lax.dot_general(sel_f, lt_strict, (((1,), (0,)), ((), ())),
                         preferred_element_type=jnp.float32)

    bt = w16.shape[0]
    kio = lax.broadcasted_iota(jnp.int32, (bt, TOPK), 1).astype(jnp.float32)
    pos8 = jnp.zeros((bt, TOPK), jnp.float32)
    w8 = jnp.zeros((bt, TOPK), jnp.float32)
    for e in range(E):
        cond = jnp.logical_and(rk[:, e:e + 1] == kio, sel[:, e:e + 1])
        pos8 = jnp.where(cond, pos16[:, e:e + 1], pos8)
        w8 = jnp.where(cond, w16[:, e:e + 1], w8)
    pos8_ref[...] = pos8.astype(jnp.int32)
    w8_ref[...] = w8

    blk_mat = lax.broadcasted_iota(jnp.int32, (E, NBP), 1).astype(
        jnp.float32) * BLK
    ge = (blk_mat >= incl_col).astype(jnp.float32)
    ones_row = jnp.ones((1, E), jnp.float32)
    be = lax.dot_general(ones_row, ge, (((1,), (0,)), ((), ())),
                         preferred_element_type=jnp.float32)
    be_ref[...] = jnp.minimum(be, float(E - 1)).astype(jnp.int32)


def _a2(prefix, w16, counts):
    bt = 512
    return pl.pallas_call(
        _a2_body,
        grid=(T // bt,),
        in_specs=[
            pl.BlockSpec((bt, E), lambda i: (i, 0)),
            pl.BlockSpec((bt, E), lambda i: (i, 0)),
            pl.BlockSpec((1, E), lambda i: (0, 0)),
        ],
        out_specs=[
            pl.BlockSpec((bt, TOPK), lambda i: (i, 0)),
            pl.BlockSpec((bt, TOPK), lambda i: (i, 0)),
            pl.BlockSpec((1, NBP), lambda i: (0, 0)),
        ],
        out_shape=[
            jax.ShapeDtypeStruct((T, TOPK), jnp.int32),
            jax.ShapeDtypeStruct((T, TOPK), jnp.float32),
            jax.ShapeDtypeStruct((1, NBP), jnp.int32),
        ],
    )(prefix, w16, counts)


def _scatter_tok_body(pos8_hbm, slot_tok_hbm, idx_v, tok_v, sem):
    wid = lax.axis_index("s") * 2 + lax.axis_index("c")
    base_p = wid * PPW
    pltpu.sync_copy(pos8_hbm.at[pl.ds(base_p, PPW)], idx_v)

    def fill(j, _):
        pair = lax.iota(jnp.int32, 16) + (base_p + j * 16)
        tok_v[pl.ds(j * 16, 16)] = lax.shift_right_logical(pair, 3)
        return 0

    lax.fori_loop(0, PPW // 16, fill, 0)
    pltpu.async_copy(tok_v, slot_tok_hbm.at[idx_v], sem).wait()


def _sc_scatter_tok(pos8_flat):
    mesh = plsc.VectorSubcoreMesh(core_axis_name="c", subcore_axis_name="s")
    kfn = pl.kernel(
        _scatter_tok_body,
        out_type=[jax.ShapeDtypeStruct((NSLOT,), jnp.int32)],
        mesh=mesh,
        scratch_types=[
            pltpu.VMEM((PPW,), jnp.int32),
            pltpu.VMEM((PPW,), jnp.int32),
            pltpu.SemaphoreType.DMA,
        ],
    )
    return kfn(pos8_flat)[0]


def _gather_x_body(x_hbm, slot_tok_hbm, xs_hbm, idxv, buf_a, buf_b, sem_a,
                   sem_b):
    wid = lax.axis_index("s") * 2 + lax.axis_index("c")
    base = wid * SPW
    pltpu.sync_copy(slot_tok_hbm.at[pl.ds(base, SPW)], idxv)

    def clamp(j, _):
        v = idxv[pl.ds(j * 16, 16)]
        idxv[pl.ds(j * 16, 16)] = jnp.minimum(
            jnp.maximum(v, jnp.int32(0)), jnp.int32(T - 1))
        return 0

    lax.fori_loop(0, SPW // 16, clamp, 0)

    G = 24
    n = SPW // G

    def gath(c, b, s):
        return pltpu.make_async_copy(
            x_hbm.at[idxv.at[pl.ds(c * G, G)]], b, s)

    gath(0, buf_a, sem_a).start()
    gath(1, buf_b, sem_b).start()

    def chunk(j, _):
        c0 = j * 2
        gath(c0, buf_a, sem_a).wait()
        pltpu.sync_copy(buf_a, xs_hbm.at[pl.ds(base + c0 * G, G)])

        @pl.when(j < n // 2 - 1)
        def _():
            gath(c0 + 2, buf_a, sem_a).start()

        gath(c0 + 1, buf_b, sem_b).wait()
        pltpu.sync_copy(buf_b, xs_hbm.at[pl.ds(base + (c0 + 1) * G, G)])

        @pl.when(j < n // 2 - 1)
        def _():
            gath(c0 + 3, buf_b, sem_b).start()

        return 0

    lax.fori_loop(0, n // 2, chunk, 0)


def _sc_gather_x(x32, slot_tok):
    mesh = plsc.VectorSubcoreMesh(core_axis_name="c", subcore_axis_name="s")
    kfn = pl.kernel(
        _gather_x_body,
        out_type=[jax.ShapeDtypeStruct((NSLOT, D), jnp.float32)],
        mesh=mesh,
        scratch_types=[
            pltpu.VMEM((SPW,), jnp.int32),
            pltpu.VMEM((24, D), jnp.float32),
            pltpu.VMEM((24, D), jnp.float32),
            pltpu.SemaphoreType.DMA,
            pltpu.SemaphoreType.DMA,
        ],
    )
    return kfn(x32, slot_tok)[0]


def _gmm_body(be_ref, xs_ref, wg_ref, wu_ref, wd_ref, y_ref):
    x = xs_ref[...].astype(jnp.bfloat16)
    g = lax.dot_general(x, wg_ref[0], (((1,), (1,)), ((), ())),
                        preferred_element_type=jnp.float32)
    u = lax.dot_general(x, wu_ref[0], (((1,), (1,)), ((), ())),
                        preferred_element_type=jnp.float32)
    h = (g * jax.nn.sigmoid(g) * u).astype(jnp.bfloat16)
    y_ref[...] = lax.dot_general(h, wd_ref[0], (((1,), (1,)), ((), ())),
                                 preferred_element_type=jnp.float32)


def _gmm(block_expert, xs, wg16, wu16, wd16):
    grid_spec = pltpu.PrefetchScalarGridSpec(
        num_scalar_prefetch=1,
        grid=(NB,),
        in_specs=[
            pl.BlockSpec((BLK, D), lambda i, be: (i, 0)),
            pl.BlockSpec((1, F, D), lambda i, be: (be[i], 0, 0)),
            pl.BlockSpec((1, F, D), lambda i, be: (be[i], 0, 0)),
            pl.BlockSpec((1, D, F), lambda i, be: (be[i], 0, 0)),
        ],
        out_specs=pl.BlockSpec((BLK, D), lambda i, be: (i, 0)),
    )
    return pl.pallas_call(
        _gmm_body,
        grid_spec=grid_spec,
        out_shape=jax.ShapeDtypeStruct((NSLOT, D), jnp.float32),
    )(block_expert, xs, wg16, wu16, wd16)


def _gather_y_body(y_hbm, pos8_hbm, yg_hbm, idxv, buf_a, buf_b, sem_a,
                   sem_b):
    wid = lax.axis_index("s") * 2 + lax.axis_index("c")
    base_p = wid * PPW
    pltpu.sync_copy(pos8_hbm.at[pl.ds(base_p, PPW)], idxv)

    G = 16
    n = PPW // G

    def gath(c, b, s):
        return pltpu.make_async_copy(
            y_hbm.at[idxv.at[pl.ds(c * G, G)]], b, s)

    gath(0, buf_a, sem_a).start()
    gath(1, buf_b, sem_b).start()

    def chunk(j, _):
        c0 = j * 2
        gath(c0, buf_a, sem_a).wait()
        pltpu.sync_copy(buf_a, yg_hbm.at[pl.ds(base_p + c0 * G, G)])

        @pl.when(j < n // 2 - 1)
        def _():
            gath(c0 + 2, buf_a, sem_a).start()

        gath(c0 + 1, buf_b, sem_b).wait()
        pltpu.sync_copy(buf_b, yg_hbm.at[pl.ds(base_p + (c0 + 1) * G, G)])

        @pl.when(j < n // 2 - 1)
        def _():
            gath(c0 + 3, buf_b, sem_b).start()

        return 0

    lax.fori_loop(0, n // 2, chunk, 0)


def _sc_gather_y(y, pos8_flat):
    mesh = plsc.VectorSubcoreMesh(core_axis_name="c", subcore_axis_name="s")
    kfn = pl.kernel(
        _gather_y_body,
        out_type=[jax.ShapeDtypeStruct((T * TOPK, D), jnp.float32)],
        mesh=mesh,
        scratch_types=[
            pltpu.VMEM((PPW,), jnp.int32),
            pltpu.VMEM((16, D), jnp.float32),
            pltpu.VMEM((16, D), jnp.float32),
            pltpu.SemaphoreType.DMA,
            pltpu.SemaphoreType.DMA,
        ],
    )
    return kfn(y, pos8_flat)[0]


def _reduce_body(yg_ref, w8_ref, s_ref, out_ref):
    w8 = w8_ref[...]
    acc = w8[:, 0:1] * yg_ref[:, 0, :]
    for k in range(1, TOPK):
        acc += w8[:, k:k + 1] * yg_ref[:, k, :]
    out_ref[...] = acc + s_ref[...]


def _reduce(yg, w8, s_out):
    bt = 256
    return pl.pallas_call(
        _reduce_body,
        grid=(T // bt,),
        in_specs=[
            pl.BlockSpec((bt, TOPK, D), lambda i: (i, 0, 0)),
            pl.BlockSpec((bt, TOPK), lambda i: (i, 0)),
            pl.BlockSpec((bt, D), lambda i: (i, 0)),
        ],
        out_specs=pl.BlockSpec((bt, D), lambda i: (i, 0)),
        out_shape=jax.ShapeDtypeStruct((T, D), jnp.float32),
    )(yg, w8, s_out)


def _shared_body(nf, x_ref, wsg_ref, wsu_ref, wsd_ref, wshg_ref,
                 out_ref, acc_ref):
    j = pl.program_id(1)
    x = x_ref[...]
    g = lax.dot_general(x, wsg_ref[...], (((1,), (1,)), ((), ())),
                        preferred_element_type=jnp.float32)
    u = lax.dot_general(x, wsu_ref[...], (((1,), (1,)), ((), ())),
                        preferred_element_type=jnp.float32)
    h = (g * jax.nn.sigmoid(g) * u).astype(jnp.bfloat16)
    part = lax.dot_general(h, wsd_ref[...], (((1,), (1,)), ((), ())),
                           preferred_element_type=jnp.float32)

    @pl.when(j == 0)
    def _():
        acc_ref[...] = part

    @pl.when(j != 0)
    def _():
        acc_ref[...] += part

    @pl.when(j == nf - 1)
    def _():
        gl = jnp.sum(x.astype(jnp.float32) * wshg_ref[...].astype(jnp.float32),
                     axis=1, keepdims=True)
        out_ref[...] = jax.nn.sigmoid(gl) * acc_ref[...]


def _shared_mlp(x16, ws_gate, ws_up, ws_down, w_shared_gate):
    f_sh = ws_gate.shape[0]
    bt = 256
    bf = 1408
    nf = f_sh // bf
    return pl.pallas_call(
        functools.partial(_shared_body, nf),
        grid=(T // bt, nf),
        in_specs=[
            pl.BlockSpec((bt, D), lambda i, j: (i, 0)),
            pl.BlockSpec((bf, D), lambda i, j: (j, 0)),
            pl.BlockSpec((bf, D), lambda i, j: (j, 0)),
            pl.BlockSpec((D, bf), lambda i, j: (0, j)),
            pl.BlockSpec((1, D), lambda i, j: (0, 0)),
        ],
        out_specs=pl.BlockSpec((bt, D), lambda i, j: (i, 0)),
        out_shape=jax.ShapeDtypeStruct((T, D), jnp.float32),
        scratch_shapes=[pltpu.VMEM((bt, D), jnp.float32)],
    )(x16, ws_gate, ws_up, ws_down, w_shared_gate)


def kernel(hidden_states, router_w, w_gate, w_up, w_down, ws_gate, ws_up,
           ws_down, w_shared_gate):
    b, s, d = hidden_states.shape
    x = hidden_states.reshape(-1, d)

    w16, prefix, counts = _router_dispatch(x, router_w)
    pos8, w8, be = _a2(prefix, w16, counts)
    pos8_flat = pos8.reshape(-1)

    x16 = x.astype(jnp.bfloat16)
    s_out = _shared_mlp(x16, ws_gate.astype(jnp.bfloat16),
                        ws_up.astype(jnp.bfloat16),
                        ws_down.astype(jnp.bfloat16),
                        w_shared_gate.astype(jnp.bfloat16))
    slot_tok = _sc_scatter_tok(pos8_flat)
    xs = _sc_gather_x(x, slot_tok)
    y = _gmm(be.reshape(-1)[:NB], xs, w_gate.astype(jnp.bfloat16),
             w_up.astype(jnp.bfloat16), w_down.astype(jnp.bfloat16))
    yg = _sc_gather_y(y, pos8_flat)
    out = _reduce(yg.reshape(T, TOPK, D), w8, s_out)
    return out.reshape(b, s, d)


# speedup vs baseline: 1.1367x; 1.1261x over previous
import functools

import jax
import jax.numpy as jnp
from jax import lax
from jax.experimental import pallas as pl
from jax.experimental.pallas import tpu as pltpu
from jax.experimental.pallas import tpu_sc as plsc

T = 4096
D = 2048
E = 16
TOPK = 8
F = 1408
BLK = 256
NSLOT = T * TOPK + E * BLK
NB = NSLOT // BLK
NBP = 256
NW = 32
TPW = T // NW
PPW = TPW * TOPK
SPW = NSLOT // NW


def _router_body(x_ref, w_ref, w16_ref, prefix_ref, counts_ref, cnt_ref):
    i = pl.program_id(0)
    x = x_ref[...]
    logits = lax.dot_general(x, w_ref[...], (((1,), (1,)), ((), ())),
                             preferred_element_type=jnp.float32)
    m = jnp.max(logits, axis=1, keepdims=True)
    ex = jnp.exp(logits - m)
    probs = ex / jnp.sum(ex, axis=1, keepdims=True)

    lane = lax.broadcasted_iota(jnp.int32, probs.shape, 1)
    sel = jnp.zeros(probs.shape, dtype=jnp.bool_)
    for _ in range(TOPK):
        cur = jnp.where(sel, -jnp.inf, probs)
        mx = jnp.max(cur, axis=1, keepdims=True)
        ismax = jnp.logical_and(cur == mx, jnp.logical_not(sel))
        first = jnp.min(jnp.where(ismax, lane, E), axis=1, keepdims=True)
        sel = jnp.logical_or(sel, lane == first)
    kept = jnp.where(sel, probs, 0.0)
    w16_ref[...] = kept / jnp.sum(kept, axis=1, keepdims=True)

    @pl.when(i == 0)
    def _():
        cnt_ref[...] = jnp.zeros_like(cnt_ref)

    self = sel.astype(jnp.float32)
    bt = self.shape[0]
    r = lax.broadcasted_iota(jnp.int32, (bt, bt), 0)
    c = lax.broadcasted_iota(jnp.int32, (bt, bt), 1)
    lt = (r > c).astype(jnp.float32)
    within = lax.dot_general(lt, self, (((1,), (0,)), ((), ())),
                             preferred_element_type=jnp.float32)
    prefix_ref[...] = (within + cnt_ref[...]).astype(jnp.int32)
    cnt_ref[...] += jnp.sum(self, axis=0, keepdims=True)
    counts_ref[...] = cnt_ref[...].astype(jnp.int32)


def _router_dispatch(x, router_w):
    bt = 512
    return pl.pallas_call(
        _router_body,
        grid=(T // bt,),
        in_specs=[
            pl.BlockSpec((bt, D), lambda i: (i, 0)),
            pl.BlockSpec((E, D), lambda i: (0, 0)),
        ],
        out_specs=[
            pl.BlockSpec((bt, E), lambda i: (i, 0)),
            pl.BlockSpec((bt, E), lambda i: (i, 0)),
            pl.BlockSpec((1, E), lambda i: (0, 0)),
        ],
        out_shape=[
            jax.ShapeDtypeStruct((T, E), jnp.float32),
            jax.ShapeDtypeStruct((T, E), jnp.int32),
            jax.ShapeDtypeStruct((1, E), jnp.int32),
        ],
        scratch_shapes=[pltpu.VMEM((1, E), jnp.float32)],
    )(x, router_w)


def _a2_body(prefix_ref, w16_ref, counts_ref, pos8_ref, w8_ref, posT_ref,
             be_ref):
    eye = (lax.broadcasted_iota(jnp.int32, (E, E), 0)
           == lax.broadcasted_iota(jnp.int32, (E, E), 1)).astype(jnp.float32)
    lt_incl = (lax.broadcasted_iota(jnp.int32, (E, E), 0)
               >= lax.broadcasted_iota(jnp.int32, (E, E), 1)).astype(
                   jnp.float32)
    lt_strict = (lax.broadcasted_iota(jnp.int32, (E, E), 0)
                 < lax.broadcasted_iota(jnp.int32, (E, E), 1)).astype(
                     jnp.float32)

    counts_row = counts_ref[...].astype(jnp.float32)
    counts_col = lax.dot_general(eye, counts_row,
                                 (((1,), (1,)), ((), ())),
                                 preferred_element_type=jnp.float32)
    aligned_col = jnp.ceil(counts_col / BLK) * BLK
    incl_col = lax.dot_general(lt_incl, aligned_col,
                               (((1,), (0,)), ((), ())),
                               preferred_element_type=jnp.float32)
    offs_col = incl_col - aligned_col
    offs_row = lax.dot_general(offs_col, eye, (((0,), (0,)), ((), ())),
                               preferred_element_type=jnp.float32)

    w16 = w16_ref[...]
    sel = w16 > 0.0
    sel_f = sel.astype(jnp.float32)
    pos16 = prefix_ref[...].astype(jnp.float32) + offs_row
    rk = lax.dot_general(sel_f, lt_strict, (((1,), (0,)), ((), ())),
                         preferred_element_type=jnp.float32)

    bt = w16.shape[0]
    kio = lax.broadcasted_iota(jnp.int32, (bt, TOPK), 1).astype(jnp.float32)
    pos8 = jnp.zeros((bt, TOPK), jnp.float32)
    w8 = jnp.zeros((bt, TOPK), jnp.float32)
    for e in range(E):
        cond = jnp.logical_and(rk[:, e:e + 1] == kio, sel[:, e:e + 1])
        pos8 = jnp.where(cond, pos16[:, e:e + 1], pos8)
        w8 = jnp.where(cond, w16[:, e:e + 1], w8)
    pos8_ref[...] = pos8.astype(jnp.int32)
    w8_ref[...] = w8
    eye_bt = (lax.broadcasted_iota(jnp.int32, (bt, bt), 0)
              == lax.broadcasted_iota(jnp.int32, (bt, bt), 1)).astype(
                  jnp.float32)
    posT_ref[...] = lax.dot_general(pos8, eye_bt, (((0,), (0,)), ((), ())),
                                    preferred_element_type=jnp.float32
                                    ).astype(jnp.int32)

    blk_mat = lax.broadcasted_iota(jnp.int32, (E, NBP), 1).astype(
        jnp.float32) * BLK
    ge = (blk_mat >= incl_col).astype(jnp.float32)
    ones_row = jnp.ones((1, E), jnp.float32)
    be = lax.dot_general(ones_row, ge, (((1,), (0,)), ((), ())),
                         preferred_element_type=jnp.float32)
    be_ref[...] = jnp.minimum(be, float(E - 1)).astype(jnp.int32)


def _a2(prefix, w16, counts):
    bt = 512
    return pl.pallas_call(
        _a2_body,
        grid=(T // bt,),
        in_specs=[
            pl.BlockSpec((bt, E), lambda i: (i, 0)),
            pl.BlockSpec((bt, E), lambda i: (i, 0)),
            pl.BlockSpec((1, E), lambda i: (0, 0)),
        ],
        out_specs=[
            pl.BlockSpec((bt, TOPK), lambda i: (i, 0)),
            pl.BlockSpec((bt, TOPK), lambda i: (i, 0)),
            pl.BlockSpec((TOPK, bt), lambda i: (0, i)),
            pl.BlockSpec((1, NBP), lambda i: (0, 0)),
        ],
        out_shape=[
            jax.ShapeDtypeStruct((T, TOPK), jnp.int32),
            jax.ShapeDtypeStruct((T, TOPK), jnp.float32),
            jax.ShapeDtypeStruct((TOPK, T), jnp.int32),
            jax.ShapeDtypeStruct((1, NBP), jnp.int32),
        ],
    )(prefix, w16, counts)


def _scatter_x_body(x_hbm, posT_hbm, xs_hbm, idxall, idx3, xbuf, sem):
    wid = lax.axis_index("s") * 2 + lax.axis_index("c")
    base_t = wid * TPW
    for k in range(TOPK):
        pltpu.sync_copy(posT_hbm.at[pl.ds(k * T + base_t, TPW)],
                        idxall.at[pl.ds(k * TPW, TPW)])

    def fill(m, _):
        idx3[m] = idxall[pl.ds(m * 16, 16)]
        return 0

    lax.fori_loop(0, TPW * TOPK // 16, fill, 0)

    def chunk(c, _):
        pltpu.sync_copy(x_hbm.at[pl.ds(base_t + c * 16, 16)], xbuf)
        descs = [
            pltpu.make_async_copy(xbuf, xs_hbm.at[idx3.at[k * (TPW // 16) + c]],
                                  sem)
            for k in range(TOPK)
        ]
        for d in descs:
            d.start()
        for d in descs:
            d.wait()
        return 0

    lax.fori_loop(0, TPW // 16, chunk, 0)


def _sc_scatter_x(x32, posT_flat):
    mesh = plsc.VectorSubcoreMesh(core_axis_name="c", subcore_axis_name="s")
    kfn = pl.kernel(
        _scatter_x_body,
        out_type=[jax.ShapeDtypeStruct((NSLOT, D), jnp.float32)],
        mesh=mesh,
        scratch_types=[
            pltpu.VMEM((TPW * TOPK,), jnp.int32),
            pltpu.VMEM((TPW * TOPK // 16, 16), jnp.int32),
            pltpu.VMEM((16, D), jnp.float32),
            pltpu.SemaphoreType.DMA,
        ],
    )
    return kfn(x32, posT_flat)[0]


def _gmm_body(be_ref, xs_ref, wg_ref, wu_ref, wd_ref, y_ref):
    x = xs_ref[...].astype(jnp.bfloat16)
    g = lax.dot_general(x, wg_ref[0], (((1,), (1,)), ((), ())),
                        preferred_element_type=jnp.float32)
    u = lax.dot_general(x, wu_ref[0], (((1,), (1,)), ((), ())),
                        preferred_element_type=jnp.float32)
    h = (g * jax.nn.sigmoid(g) * u).astype(jnp.bfloat16)
    y_ref[...] = lax.dot_general(h, wd_ref[0], (((1,), (1,)), ((), ())),
                                 preferred_element_type=jnp.float32)


def _gmm(block_expert, xs, wg16, wu16, wd16):
    grid_spec = pltpu.PrefetchScalarGridSpec(
        num_scalar_prefetch=1,
        grid=(NB,),
        in_specs=[
            pl.BlockSpec((BLK, D), lambda i, be: (i, 0)),
            pl.BlockSpec((1, F, D), lambda i, be: (be[i], 0, 0)),
            pl.BlockSpec((1, F, D), lambda i, be: (be[i], 0, 0)),
            pl.BlockSpec((1, D, F), lambda i, be: (be[i], 0, 0)),
        ],
        out_specs=pl.BlockSpec((BLK, D), lambda i, be: (i, 0)),
    )
    return pl.pallas_call(
        _gmm_body,
        grid_spec=grid_spec,
        out_shape=jax.ShapeDtypeStruct((NSLOT, D), jnp.float32),
    )(block_expert, xs, wg16, wu16, wd16)


def _gather_y_body(y_hbm, pos8_hbm, yg_hbm, idxv, buf_a, buf_b, sem_a,
                   sem_b):
    wid = lax.axis_index("s") * 2 + lax.axis_index("c")
    base_p = wid * PPW
    pltpu.sync_copy(pos8_hbm.at[pl.ds(base_p, PPW)], idxv)

    G = 16
    n = PPW // G

    def gath(c, b, s):
        return pltpu.make_async_copy(
            y_hbm.at[idxv.at[pl.ds(c * G, G)]], b, s)

    gath(0, buf_a, sem_a).start()
    gath(1, buf_b, sem_b).start()

    def chunk(j, _):
        c0 = j * 2
        gath(c0, buf_a, sem_a).wait()
        pltpu.sync_copy(buf_a, yg_hbm.at[pl.ds(base_p + c0 * G, G)])

        @pl.when(j < n // 2 - 1)
        def _():
            gath(c0 + 2, buf_a, sem_a).start()

        gath(c0 + 1, buf_b, sem_b).wait()
        pltpu.sync_copy(buf_b, yg_hbm.at[pl.ds(base_p + (c0 + 1) * G, G)])

        @pl.when(j < n // 2 - 1)
        def _():
            gath(c0 + 3, buf_b, sem_b).start()

        return 0

    lax.fori_loop(0, n // 2, chunk, 0)


def _sc_gather_y(y, pos8_flat):
    mesh = plsc.VectorSubcoreMesh(core_axis_name="c", subcore_axis_name="s")
    kfn = pl.kernel(
        _gather_y_body,
        out_type=[jax.ShapeDtypeStruct((T * TOPK, D), jnp.float32)],
        mesh=mesh,
        scratch_types=[
            pltpu.VMEM((PPW,), jnp.int32),
            pltpu.VMEM((16, D), jnp.float32),
            pltpu.VMEM((16, D), jnp.float32),
            pltpu.SemaphoreType.DMA,
            pltpu.SemaphoreType.DMA,
        ],
    )
    return kfn(y, pos8_flat)[0]


def _reduce_body(yg_ref, w8_ref, s_ref, out_ref):
    w8 = w8_ref[...]
    acc = w8[:, 0:1] * yg_ref[:, 0, :]
    for k in range(1, TOPK):
        acc += w8[:, k:k + 1] * yg_ref[:, k, :]
    out_ref[...] = acc + s_ref[...]


def _reduce(yg, w8, s_out):
    bt = 256
    return pl.pallas_call(
        _reduce_body,
        grid=(T // bt,),
        in_specs=[
            pl.BlockSpec((bt, TOPK, D), lambda i: (i, 0, 0)),
            pl.BlockSpec((bt, TOPK), lambda i: (i, 0)),
            pl.BlockSpec((bt, D), lambda i: (i, 0)),
        ],
        out_specs=pl.BlockSpec((bt, D), lambda i: (i, 0)),
        out_shape=jax.ShapeDtypeStruct((T, D), jnp.float32),
    )(yg, w8, s_out)


def _shared_body(nf, x_ref, wsg_ref, wsu_ref, wsd_ref, wshg_ref,
                 out_ref, acc_ref):
    j = pl.program_id(1)
    x = x_ref[...]
    g = lax.dot_general(x, wsg_ref[...], (((1,), (1,)), ((), ())),
                        preferred_element_type=jnp.float32)
    u = lax.dot_general(x, wsu_ref[...], (((1,), (1,)), ((), ())),
                        preferred_element_type=jnp.float32)
    h = (g * jax.nn.sigmoid(g) * u).astype(jnp.bfloat16)
    part = lax.dot_general(h, wsd_ref[...], (((1,), (1,)), ((), ())),
                           preferred_element_type=jnp.float32)

    @pl.when(j == 0)
    def _():
        acc_ref[...] = part

    @pl.when(j != 0)
    def _():
        acc_ref[...] += part

    @pl.when(j == nf - 1)
    def _():
        gl = jnp.sum(x.astype(jnp.float32) * wshg_ref[...].astype(jnp.float32),
                     axis=1, keepdims=True)
        out_ref[...] = jax.nn.sigmoid(gl) * acc_ref[...]


def _shared_mlp(x16, ws_gate, ws_up, ws_down, w_shared_gate):
    f_sh = ws_gate.shape[0]
    bt = 256
    bf = 1408
    nf = f_sh // bf
    return pl.pallas_call(
        functools.partial(_shared_body, nf),
        grid=(T // bt, nf),
        in_specs=[
            pl.BlockSpec((bt, D), lambda i, j: (i, 0)),
            pl.BlockSpec((bf, D), lambda i, j: (j, 0)),
            pl.BlockSpec((bf, D), lambda i, j: (j, 0)),
            pl.BlockSpec((D, bf), lambda i, j: (0, j)),
            pl.BlockSpec((1, D), lambda i, j: (0, 0)),
        ],
        out_specs=pl.BlockSpec((bt, D), lambda i, j: (i, 0)),
        out_shape=jax.ShapeDtypeStruct((T, D), jnp.float32),
        scratch_shapes=[pltpu.VMEM((bt, D), jnp.float32)],
    )(x16, ws_gate, ws_up, ws_down, w_shared_gate)


def kernel(hidden_states, router_w, w_gate, w_up, w_down, ws_gate, ws_up,
           ws_down, w_shared_gate):
    b, s, d = hidden_states.shape
    x = hidden_states.reshape(-1, d)

    w16, prefix, counts = _router_dispatch(x, router_w)
    pos8, w8, posT, be = _a2(prefix, w16, counts)
    pos8_flat = pos8.reshape(-1)

    x16 = x.astype(jnp.bfloat16)
    s_out = _shared_mlp(x16, ws_gate.astype(jnp.bfloat16),
                        ws_up.astype(jnp.bfloat16),
                        ws_down.astype(jnp.bfloat16),
                        w_shared_gate.astype(jnp.bfloat16))
    xs = _sc_scatter_x(x, posT.reshape(-1))
    y = _gmm(be.reshape(-1)[:NB], xs, w_gate.astype(jnp.bfloat16),
             w_up.astype(jnp.bfloat16), w_down.astype(jnp.bfloat16))
    yg = _sc_gather_y(y, pos8_flat)
    out = _reduce(yg.reshape(T, TOPK, D), w8, s_out)
    return out.reshape(b, s, d)
